# baseline (speedup 1.0000x reference)
import functools

import jax
import jax.numpy as jnp
from jax import lax
from jax.experimental import pallas as pl
from jax.experimental.pallas import tpu as pltpu
from jax.experimental.pallas import tpu_sc as plsc

N = 10000
F = 128
C = 64
NPG = 100
G = N // NPG
E = 320000
H = 256

NUM_CORES = 2
NUM_SUBCORES = 16
NW = NUM_CORES * NUM_SUBCORES
EPW = E // NW
CHUNK = 80
K = EPW // CHUNK
NBUF = 4
ROWS_PER_TILE = 624
TAIL_ROWS = N - ROWS_PER_TILE * NUM_SUBCORES
ZROWS = 104


def _msg_body(x_ref, wc_ref, bc_ref, m_ref):
    m_ref[...] = (
        jnp.dot(x_ref[...], wc_ref[...], preferred_element_type=jnp.float32)
        + bc_ref[...]
    )


_msg_call = pl.pallas_call(
    _msg_body,
    out_shape=jax.ShapeDtypeStruct((N, C), jnp.float32),
)


def _sc_scatter_body(edge_hbm, m_hbm, out_hbm,
                     src_v, dst_stage, dst_v,
                     gbuf0, gbuf1, gbuf2, gbuf3, zbuf, agg_sh,
                     gsem0, gsem1, gsem2, gsem3,
                     ssem0, ssem1, ssem2, ssem3):
    cid = lax.axis_index("c")
    sid = lax.axis_index("s")
    wid = cid * NUM_SUBCORES + sid
    base = wid * EPW
    gbuf = [gbuf0, gbuf1, gbuf2, gbuf3]
    gsem = [gsem0, gsem1, gsem2, gsem3]
    ssem = [ssem0, ssem1, ssem2, ssem3]

    pltpu.async_copy(edge_hbm.at[0, pl.ds(base, EPW)], src_v, gsem0)
    pltpu.async_copy(edge_hbm.at[1, pl.ds(base, EPW)], dst_stage, gsem1)

    zero16 = jnp.zeros((16,), jnp.float32)

    def _zrow(r, carry):
        for q in range(C // 16):
            zbuf[r, pl.ds(q * 16, 16)] = zero16
        return carry

    lax.fori_loop(0, ZROWS, _zrow, 0)

    row0 = sid * ROWS_PER_TILE
    for b in range(ROWS_PER_TILE // ZROWS):
        pltpu.sync_copy(zbuf, agg_sh.at[pl.ds(row0 + b * ZROWS, ZROWS)])

    @pl.when(sid == NUM_SUBCORES - 1)
    def _zero_tail():
        pltpu.sync_copy(
            zbuf.at[pl.ds(0, TAIL_ROWS)],
            agg_sh.at[pl.ds(N - TAIL_ROWS, TAIL_ROWS)],
        )

    pltpu.make_async_copy(edge_hbm.at[0, pl.ds(base, EPW)], src_v, gsem0).wait()
    pltpu.make_async_copy(
        edge_hbm.at[1, pl.ds(base, EPW)], dst_stage, gsem1).wait()

    def _prow(j, carry):
        for q in range(CHUNK // 16):
            dst_v[j, pl.ds(q * 16, 16)] = dst_stage[pl.ds(j * CHUNK + q * 16, 16)]
        return carry

    lax.fori_loop(0, K, _prow, 0)

    plsc.subcore_barrier()

    def _gather(j, b):
        pltpu.async_copy(
            m_hbm.at[src_v.at[pl.ds(j * CHUNK, CHUNK)]], gbuf[b], gsem[b])

    def _gather_wait(j, b):
        pltpu.make_async_copy(
            m_hbm.at[src_v.at[pl.ds(j * CHUNK, CHUNK)]], gbuf[b], gsem[b]).wait()

    for b in range(NBUF):
        _gather(b, b)

    def _round(jj, carry):
        for b in range(NBUF):
            j = NBUF * jj + b
            _gather_wait(j, b)
            pltpu.async_copy(gbuf[b], agg_sh.at[dst_v.at[j]], ssem[b], add=True)

            @pl.when(j + NBUF < K)
            def _refill():
                pltpu.make_async_copy(
                    gbuf[b], agg_sh.at[dst_v.at[j]], ssem[b]).wait()
                _gather(j + NBUF, b)
        return carry

    lax.fori_loop(0, (K - 1) // NBUF, _round, 0)

    _gather_wait(K - 1, 0)
    pltpu.sync_copy(gbuf[0], agg_sh.at[dst_v.at[K - 1]], add=True)
    for b in range(1, NBUF):
        pltpu.make_async_copy(
            gbuf[b], agg_sh.at[dst_v.at[K - NBUF - 1 + b]], ssem[b]).wait()

    plsc.subcore_barrier()

    pltpu.sync_copy(
        agg_sh.at[pl.ds(row0, ROWS_PER_TILE)],
        out_hbm.at[cid, pl.ds(row0, ROWS_PER_TILE)],
    )

    @pl.when(sid == NUM_SUBCORES - 1)
    def _write_tail():
        pltpu.sync_copy(
            agg_sh.at[pl.ds(N - TAIL_ROWS, TAIL_ROWS)],
            out_hbm.at[cid, pl.ds(N - TAIL_ROWS, TAIL_ROWS)],
        )


def _mlp_body(p_ref, w1_ref, b1_ref, w2_ref, b2_ref, w3_ref, b3_ref, o_ref):
    s = p_ref[0] + p_ref[1]
    h = jnp.where(s > 0.0, s, jnp.exp(jnp.minimum(s, 0.0)) - 1.0)
    z1 = jnp.dot(h, w1_ref[...], preferred_element_type=jnp.float32) + b1_ref[...]
    z1 = jnp.maximum(z1, 0.0)
    z2 = jnp.dot(z1, w2_ref[...], preferred_element_type=jnp.float32) + b2_ref[...]
    z2 = jnp.maximum(z2, 0.0)
    z3 = jnp.dot(z2, w3_ref[...], preferred_element_type=jnp.float32) + b3_ref[...]
    o_ref[...] = 1.0 / (1.0 + jnp.exp(-z3))


_mlp_call = pl.pallas_call(
    _mlp_body,
    out_shape=jax.ShapeDtypeStruct((G, 1), jnp.float32),
)


@functools.cache
def _sc_scatter_call():
    mesh = plsc.VectorSubcoreMesh(
        core_axis_name="c", subcore_axis_name="s",
        num_cores=NUM_CORES, num_subcores=NUM_SUBCORES,
    )
    return pl.kernel(
        _sc_scatter_body,
        out_type=jax.ShapeDtypeStruct((NUM_CORES, N, C), jnp.float32),
        mesh=mesh,
        scratch_types=(
            [
                pltpu.VMEM((EPW,), jnp.int32),
                pltpu.VMEM((EPW,), jnp.int32),
                pltpu.VMEM((K, CHUNK), jnp.int32),
            ]
            + [pltpu.VMEM((CHUNK, C), jnp.float32)] * NBUF
            + [
                pltpu.VMEM((ZROWS, C), jnp.float32),
                pltpu.VMEM_SHARED((N, C), jnp.float32),
            ]
            + [pltpu.SemaphoreType.DMA] * (2 * NBUF)
        ),
        compiler_params=pltpu.CompilerParams(use_tc_tiling_on_sc=False),
    )


def kernel(x, edge_index, i, Wc, bc, W1, b1, W2, b2, W3, b3):
    del i
    m = _msg_call(x, Wc, bc.reshape(1, C))
    partials = _sc_scatter_call()(edge_index, m)
    p = partials.reshape(NUM_CORES, G, NPG * C)
    out = _mlp_call(p, W1, b1.reshape(1, -1), W2, b2.reshape(1, -1),
                    W3, b3.reshape(1, 1))
    return out


# speedup vs baseline: 1.0362x; 1.0362x over previous
import functools

import jax
import jax.numpy as jnp
from jax import lax
from jax.experimental import pallas as pl
from jax.experimental.pallas import tpu as pltpu
from jax.experimental.pallas import tpu_sc as plsc

N = 10000
F = 128
C = 64
NPG = 100
G = N // NPG
E = 320000
H = 256

NUM_CORES = 2
NUM_SUBCORES = 16
NW = NUM_CORES * NUM_SUBCORES
EPW = E // NW
CHUNK = 80
K = EPW // CHUNK
NBUF = 4
ROWS_PER_TILE = 624
TAIL_ROWS = N - ROWS_PER_TILE * NUM_SUBCORES
ZROWS = 104


def _msg_body(x_ref, wc_ref, bc_ref, m_ref):
    m_lo = jnp.dot(x_ref[0:N // 2], wc_ref[...],
                   preferred_element_type=jnp.float32) + bc_ref[...]
    m_hi = jnp.dot(x_ref[N // 2:N], wc_ref[...],
                   preferred_element_type=jnp.float32) + bc_ref[...]
    m_ref[...] = jnp.concatenate([m_lo, m_hi], axis=1)


_msg_call = pl.pallas_call(
    _msg_body,
    out_shape=jax.ShapeDtypeStruct((N // 2, 2 * C), jnp.float32),
)


def _sc_scatter_body(edge_hbm, m_hbm, out_hbm,
                     src_v, fsrc_v, dst_stage, dst_v,
                     gbuf0, gbuf1, gbuf2, gbuf3, zbuf, agg_sh,
                     gsem0, gsem1, gsem2, gsem3,
                     ssem0, ssem1, ssem2, ssem3):
    cid = lax.axis_index("c")
    sid = lax.axis_index("s")
    wid = cid * NUM_SUBCORES + sid
    base = wid * EPW
    gbuf = [gbuf0, gbuf1, gbuf2, gbuf3]
    gsem = [gsem0, gsem1, gsem2, gsem3]
    ssem = [ssem0, ssem1, ssem2, ssem3]

    pltpu.async_copy(edge_hbm.at[0, pl.ds(base, EPW)], src_v, gsem0)
    pltpu.async_copy(edge_hbm.at[1, pl.ds(base, EPW)], dst_stage, gsem1)

    zero16 = jnp.zeros((16,), jnp.float32)

    def _zrow(r, carry):
        for q in range(C // 16):
            zbuf[r, pl.ds(q * 16, 16)] = zero16
        return carry

    lax.fori_loop(0, ZROWS, _zrow, 0)

    row0 = sid * ROWS_PER_TILE
    for b in range(ROWS_PER_TILE // ZROWS):
        pltpu.sync_copy(zbuf, agg_sh.at[pl.ds(row0 + b * ZROWS, ZROWS)])

    @pl.when(sid == NUM_SUBCORES - 1)
    def _zero_tail():
        pltpu.sync_copy(
            zbuf.at[pl.ds(0, TAIL_ROWS)],
            agg_sh.at[pl.ds(N - TAIL_ROWS, TAIL_ROWS)],
        )

    pltpu.make_async_copy(edge_hbm.at[0, pl.ds(base, EPW)], src_v, gsem0).wait()
    pltpu.make_async_copy(
        edge_hbm.at[1, pl.ds(base, EPW)], dst_stage, gsem1).wait()

    def _prow(j, carry):
        for q in range(CHUNK // 16):
            dst_v[j, pl.ds(q * 16, 16)] = dst_stage[pl.ds(j * CHUNK + q * 16, 16)]
        return carry

    lax.fori_loop(0, K, _prow, 0)

    plsc.subcore_barrier()

    def _xform(j):
        for q in range(CHUNK // 16):
            off = j * CHUNK + q * 16
            s = src_v[pl.ds(off, 16)]
            fsrc_v[pl.ds(off, 16)] = jnp.where(
                s < N // 2, 2 * s, 2 * s - (N - 1))

    def _gather(j, b):
        pltpu.async_copy(
            m_hbm.at[fsrc_v.at[pl.ds(j * CHUNK, CHUNK)]], gbuf[b], gsem[b])

    def _gather_wait(j, b):
        pltpu.make_async_copy(
            m_hbm.at[fsrc_v.at[pl.ds(j * CHUNK, CHUNK)]], gbuf[b], gsem[b]).wait()

    for b in range(NBUF):
        _xform(b)
        _gather(b, b)

    def _round(jj, carry):
        for b in range(NBUF):
            j = NBUF * jj + b

            @pl.when(j + NBUF < K)
            def _prep():
                _xform(j + NBUF)

            _gather_wait(j, b)
            pltpu.async_copy(gbuf[b], agg_sh.at[dst_v.at[j]], ssem[b], add=True)

            @pl.when(j + NBUF < K)
            def _refill():
                pltpu.make_async_copy(
                    gbuf[b], agg_sh.at[dst_v.at[j]], ssem[b]).wait()
                _gather(j + NBUF, b)
        return carry

    lax.fori_loop(0, (K - 1) // NBUF, _round, 0)

    _gather_wait(K - 1, 0)
    pltpu.sync_copy(gbuf[0], agg_sh.at[dst_v.at[K - 1]], add=True)
    for b in range(1, NBUF):
        pltpu.make_async_copy(
            gbuf[b], agg_sh.at[dst_v.at[K - NBUF - 1 + b]], ssem[b]).wait()

    plsc.subcore_barrier()

    pltpu.sync_copy(
        agg_sh.at[pl.ds(row0, ROWS_PER_TILE)],
        out_hbm.at[cid, pl.ds(row0, ROWS_PER_TILE)],
    )

    @pl.when(sid == NUM_SUBCORES - 1)
    def _write_tail():
        pltpu.sync_copy(
            agg_sh.at[pl.ds(N - TAIL_ROWS, TAIL_ROWS)],
            out_hbm.at[cid, pl.ds(N - TAIL_ROWS, TAIL_ROWS)],
        )


def _mlp_body(p_ref, w1_ref, b1_ref, w2_ref, b2_ref, w3_ref, b3_ref, o_ref):
    s = p_ref[0] + p_ref[1]
    h = jnp.where(s > 0.0, s, jnp.exp(jnp.minimum(s, 0.0)) - 1.0)
    z1 = jnp.dot(h, w1_ref[...], preferred_element_type=jnp.float32) + b1_ref[...]
    z1 = jnp.maximum(z1, 0.0)
    z2 = jnp.dot(z1, w2_ref[...], preferred_element_type=jnp.float32) + b2_ref[...]
    z2 = jnp.maximum(z2, 0.0)
    z3 = jnp.dot(z2, w3_ref[...], preferred_element_type=jnp.float32) + b3_ref[...]
    o_ref[...] = 1.0 / (1.0 + jnp.exp(-z3))


_mlp_call = pl.pallas_call(
    _mlp_body,
    out_shape=jax.ShapeDtypeStruct((G, 1), jnp.float32),
)


@functools.cache
def _sc_scatter_call():
    mesh = plsc.VectorSubcoreMesh(
        core_axis_name="c", subcore_axis_name="s",
        num_cores=NUM_CORES, num_subcores=NUM_SUBCORES,
    )
    return pl.kernel(
        _sc_scatter_body,
        out_type=jax.ShapeDtypeStruct((NUM_CORES, N, C), jnp.float32),
        mesh=mesh,
        scratch_types=(
            [
                pltpu.VMEM((EPW,), jnp.int32),
                pltpu.VMEM((EPW,), jnp.int32),
                pltpu.VMEM((EPW,), jnp.int32),
                pltpu.VMEM((K, CHUNK), jnp.int32),
            ]
            + [pltpu.VMEM((CHUNK, C), jnp.float32)] * NBUF
            + [
                pltpu.VMEM((ZROWS, C), jnp.float32),
                pltpu.VMEM_SHARED((N, C), jnp.float32),
            ]
            + [pltpu.SemaphoreType.DMA] * (2 * NBUF)
        ),
        compiler_params=pltpu.CompilerParams(use_tc_tiling_on_sc=False),
    )


def kernel(x, edge_index, i, Wc, bc, W1, b1, W2, b2, W3, b3):
    del i
    m = _msg_call(x, Wc, bc.reshape(1, C)).reshape(N, C)
    partials = _sc_scatter_call()(edge_index, m)
    p = partials.reshape(NUM_CORES, G, NPG * C)
    out = _mlp_call(p, W1, b1.reshape(1, -1), W2, b2.reshape(1, -1),
                    W3, b3.reshape(1, 1))
    return out


# speedup vs baseline: 1.0373x; 1.0011x over previous
import functools

import jax
import jax.numpy as jnp
from jax import lax
from jax.experimental import pallas as pl
from jax.experimental.pallas import tpu as pltpu
from jax.experimental.pallas import tpu_sc as plsc

N = 10000
F = 128
C = 64
NPG = 100
G = N // NPG
E = 320000
H = 256

NUM_CORES = 2
NUM_SUBCORES = 16
NW = NUM_CORES * NUM_SUBCORES
EPW = E // NW
CHUNK = 80
K = EPW // CHUNK
NBUF = 4
ROWS_PER_TILE = 624
TAIL_ROWS = N - ROWS_PER_TILE * NUM_SUBCORES
ZROWS = 104


def _msg_body(x_ref, wc_ref, bc_ref, m_ref):
    m_lo = jnp.dot(x_ref[0:N // 2], wc_ref[...],
                   preferred_element_type=jnp.float32) + bc_ref[...]
    m_hi = jnp.dot(x_ref[N // 2:N], wc_ref[...],
                   preferred_element_type=jnp.float32) + bc_ref[...]
    m_ref[...] = jnp.concatenate([m_lo, m_hi], axis=1)


_msg_call = pl.pallas_call(
    _msg_body,
    out_shape=jax.ShapeDtypeStruct((N // 2, 2 * C), jnp.float32),
)


def _sc_scatter_body(edge_hbm, m_hbm, out_hbm,
                     src_v, fsrc_v, dst_stage, dst_v,
                     gbuf0, gbuf1, gbuf2, gbuf3, zbuf, agg_sh,
                     gsem0, gsem1, gsem2, gsem3,
                     ssem0, ssem1, ssem2, ssem3):
    cid = lax.axis_index("c")
    sid = lax.axis_index("s")
    wid = cid * NUM_SUBCORES + sid
    base = wid * EPW
    gbuf = [gbuf0, gbuf1, gbuf2, gbuf3]
    gsem = [gsem0, gsem1, gsem2, gsem3]
    ssem = [ssem0, ssem1, ssem2, ssem3]

    pltpu.async_copy(edge_hbm.at[0, pl.ds(base, EPW)], src_v, ssem0)
    pltpu.async_copy(edge_hbm.at[1, pl.ds(base, EPW)], dst_stage, ssem1)
    pltpu.make_async_copy(edge_hbm.at[0, pl.ds(base, EPW)], src_v, ssem0).wait()

    def _xform(j):
        for q in range(CHUNK // 16):
            off = j * CHUNK + q * 16
            s = src_v[pl.ds(off, 16)]
            fsrc_v[pl.ds(off, 16)] = jnp.where(
                s < N // 2, 2 * s, 2 * s - (N - 1))

    def _gather(j, b):
        pltpu.async_copy(
            m_hbm.at[fsrc_v.at[pl.ds(j * CHUNK, CHUNK)]], gbuf[b], gsem[b])

    def _gather_wait(j, b):
        pltpu.make_async_copy(
            m_hbm.at[fsrc_v.at[pl.ds(j * CHUNK, CHUNK)]], gbuf[b], gsem[b]).wait()

    for b in range(NBUF):
        _xform(b)
        _gather(b, b)

    zero16 = jnp.zeros((16,), jnp.float32)

    def _zrow(r, carry):
        for q in range(C // 16):
            zbuf[r, pl.ds(q * 16, 16)] = zero16
        return carry

    lax.fori_loop(0, ZROWS, _zrow, 0)

    row0 = sid * ROWS_PER_TILE
    for b in range(ROWS_PER_TILE // ZROWS):
        pltpu.sync_copy(zbuf, agg_sh.at[pl.ds(row0 + b * ZROWS, ZROWS)])

    @pl.when(sid == NUM_SUBCORES - 1)
    def _zero_tail():
        pltpu.sync_copy(
            zbuf.at[pl.ds(0, TAIL_ROWS)],
            agg_sh.at[pl.ds(N - TAIL_ROWS, TAIL_ROWS)],
        )

    pltpu.make_async_copy(
        edge_hbm.at[1, pl.ds(base, EPW)], dst_stage, ssem1).wait()

    def _prow(j, carry):
        for q in range(CHUNK // 16):
            dst_v[j, pl.ds(q * 16, 16)] = dst_stage[pl.ds(j * CHUNK + q * 16, 16)]
        return carry

    lax.fori_loop(0, K, _prow, 0)

    plsc.subcore_barrier()

    def _round(jj, carry):
        for b in range(NBUF):
            j = NBUF * jj + b

            @pl.when(j + NBUF < K)
            def _prep():
                _xform(j + NBUF)

            _gather_wait(j, b)
            pltpu.async_copy(gbuf[b], agg_sh.at[dst_v.at[j]], ssem[b], add=True)

            @pl.when(j + NBUF < K)
            def _refill():
                pltpu.make_async_copy(
                    gbuf[b], agg_sh.at[dst_v.at[j]], ssem[b]).wait()
                _gather(j + NBUF, b)
        return carry

    lax.fori_loop(0, (K - 1) // NBUF, _round, 0)

    _gather_wait(K - 1, 0)
    pltpu.sync_copy(gbuf[0], agg_sh.at[dst_v.at[K - 1]], add=True)
    for b in range(1, NBUF):
        pltpu.make_async_copy(
            gbuf[b], agg_sh.at[dst_v.at[K - NBUF - 1 + b]], ssem[b]).wait()

    plsc.subcore_barrier()

    pltpu.sync_copy(
        agg_sh.at[pl.ds(row0, ROWS_PER_TILE)],
        out_hbm.at[cid, pl.ds(row0, ROWS_PER_TILE)],
    )

    @pl.when(sid == NUM_SUBCORES - 1)
    def _write_tail():
        pltpu.sync_copy(
            agg_sh.at[pl.ds(N - TAIL_ROWS, TAIL_ROWS)],
            out_hbm.at[cid, pl.ds(N - TAIL_ROWS, TAIL_ROWS)],
        )


def _mlp_body(p_ref, w1_ref, b1_ref, w2_ref, b2_ref, w3_ref, b3_ref, o_ref):
    s = p_ref[0] + p_ref[1]
    h = jnp.where(s > 0.0, s, jnp.exp(jnp.minimum(s, 0.0)) - 1.0)
    z1 = jnp.dot(h, w1_ref[...], preferred_element_type=jnp.float32) + b1_ref[...]
    z1 = jnp.maximum(z1, 0.0)
    z2 = jnp.dot(z1, w2_ref[...], preferred_element_type=jnp.float32) + b2_ref[...]
    z2 = jnp.maximum(z2, 0.0)
    z3 = jnp.dot(z2, w3_ref[...], preferred_element_type=jnp.float32) + b3_ref[...]
    o_ref[...] = 1.0 / (1.0 + jnp.exp(-z3))


_mlp_call = pl.pallas_call(
    _mlp_body,
    out_shape=jax.ShapeDtypeStruct((G, 1), jnp.float32),
)


@functools.cache
def _sc_scatter_call():
    mesh = plsc.VectorSubcoreMesh(
        core_axis_name="c", subcore_axis_name="s",
        num_cores=NUM_CORES, num_subcores=NUM_SUBCORES,
    )
    return pl.kernel(
        _sc_scatter_body,
        out_type=jax.ShapeDtypeStruct((NUM_CORES, N, C), jnp.float32),
        mesh=mesh,
        scratch_types=(
            [
                pltpu.VMEM((EPW,), jnp.int32),
                pltpu.VMEM((EPW,), jnp.int32),
                pltpu.VMEM((EPW,), jnp.int32),
                pltpu.VMEM((K, CHUNK), jnp.int32),
            ]
            + [pltpu.VMEM((CHUNK, C), jnp.float32)] * NBUF
            + [
                pltpu.VMEM((ZROWS, C), jnp.float32),
                pltpu.VMEM_SHARED((N, C), jnp.float32),
            ]
            + [pltpu.SemaphoreType.DMA] * (2 * NBUF)
        ),
        compiler_params=pltpu.CompilerParams(use_tc_tiling_on_sc=False),
    )


def kernel(x, edge_index, i, Wc, bc, W1, b1, W2, b2, W3, b3):
    del i
    m = _msg_call(x, Wc, bc.reshape(1, C)).reshape(N, C)
    partials = _sc_scatter_call()(edge_index, m)
    p = partials.reshape(NUM_CORES, G, NPG * C)
    out = _mlp_call(p, W1, b1.reshape(1, -1), W2, b2.reshape(1, -1),
                    W3, b3.reshape(1, 1))
    return out
